# baseline (speedup 1.0000x reference)
import functools

import jax
import jax.numpy as jnp
from jax import lax
from jax.experimental import pallas as pl
from jax.experimental.pallas import tpu as pltpu
from jax.experimental.pallas import tpu_sc as plsc

HIDDEN = 32
CHUNK = 512
NC, NS = 2, 16
NW = NC * NS


def _gather_call(table_flat, idx3):
    nw, nb, w = idx3.shape
    batch = nw * nb * w
    rows_per_worker = nb * w
    mesh = plsc.VectorSubcoreMesh(core_axis_name="c", subcore_axis_name="s")

    @functools.partial(
        pl.kernel,
        out_type=jax.ShapeDtypeStruct((batch * HIDDEN,), table_flat.dtype),
        mesh=mesh,
        scratch_types=[
            pltpu.VMEM((2, w), jnp.int32),
            pltpu.VMEM((CHUNK * HIDDEN,), jnp.float32),
            pltpu.VMEM((CHUNK * HIDDEN,), jnp.float32),
            pltpu.SemaphoreType.DMA,
            pltpu.SemaphoreType.DMA,
            pltpu.SemaphoreType.DMA,
            pltpu.SemaphoreType.DMA,
            pltpu.SemaphoreType.DMA,
        ],
    )
    def k(table_hbm, idx_hbm, out_hbm, idx_vm, rb0, rb1, si0, sg0, sg1, sw0, sw1):
        wid = lax.axis_index("s") * NC + lax.axis_index("c")
        base = wid * rows_per_worker * HIDDEN

        bufs = (rb0, rb1)
        gsems = (sg0, sg1)
        wsems = (sw0, sw1)

        def idx_load(j, t):
            pltpu.make_async_copy(
                idx_hbm.at[wid, j], idx_vm.at[t], si0
            ).start()

        def idx_wait(t):
            pltpu.make_async_copy(
                idx_hbm.at[wid, 0], idx_vm.at[t], si0
            ).wait()

        def gather_chunk(t):
            @pl.loop(0, CHUNK, step=16)
            def _(r):
                v = idx_vm[t, pl.ds(r, 16)]
                for l in range(16):
                    pltpu.make_async_copy(
                        table_hbm.at[pl.ds(v[l] * HIDDEN, HIDDEN)],
                        bufs[t].at[pl.ds((r + l) * HIDDEN, HIDDEN)],
                        gsems[t],
                    ).start()

        def gather_drain(t):
            pltpu.make_async_copy(
                table_hbm.at[pl.ds(0, CHUNK * HIDDEN)], bufs[t], gsems[t]
            ).wait()

        def write_start(j, t):
            pltpu.make_async_copy(
                bufs[t],
                out_hbm.at[pl.ds(base + j * CHUNK * HIDDEN, CHUNK * HIDDEN)],
                wsems[t],
            ).start()

        def write_wait(t):
            pltpu.make_async_copy(
                bufs[t],
                out_hbm.at[pl.ds(base, CHUNK * HIDDEN)],
                wsems[t],
            ).wait()

        idx_load(0, 0)
        idx_load(1, 1)

        @pl.loop(0, nb, step=2)
        def _(j):
            for t in range(2):
                idx_wait(t)

                @pl.when(j > 0)
                def _():
                    write_wait(t)

                gather_chunk(t)

            for t in range(2):
                gather_drain(t)
                write_start(j + t, t)

            @pl.when(j + 2 < nb)
            def _():
                for t in range(2):
                    idx_load(j + 2 + t, t)

        for t in range(2):
            write_wait(t)

    return k(table_flat, idx3)


def kernel(input_ids, table):
    batch = input_ids.size
    table_flat = table.reshape(-1)
    idx3 = input_ids.astype(jnp.int32).reshape(NW, batch // (NW * CHUNK), CHUNK)
    out = _gather_call(table_flat, idx3)
    return out.reshape(*input_ids.shape, HIDDEN)


# speedup vs baseline: 1.1693x; 1.1693x over previous
import functools

import jax
import jax.numpy as jnp
from jax import lax
from jax.experimental import pallas as pl
from jax.experimental.pallas import tpu as pltpu
from jax.experimental.pallas import tpu_sc as plsc

HIDDEN = 32
SEQ = 50
BPC = 8
CHUNK = BPC * SEQ
NC, NS = 2, 16
NW = NC * NS


def _gather_call(table, idx3, n_batch):
    nw, nb, w = idx3.shape
    batches_per_worker = nb * BPC
    mesh = plsc.VectorSubcoreMesh(core_axis_name="c", subcore_axis_name="s")

    @functools.partial(
        pl.kernel,
        out_type=jax.ShapeDtypeStruct((n_batch, SEQ, HIDDEN), table.dtype),
        mesh=mesh,
        scratch_types=[
            pltpu.VMEM((2, w), jnp.int32),
            pltpu.VMEM((CHUNK, HIDDEN), jnp.float32),
            pltpu.VMEM((CHUNK, HIDDEN), jnp.float32),
            pltpu.SemaphoreType.DMA,
            pltpu.SemaphoreType.DMA,
            pltpu.SemaphoreType.DMA,
            pltpu.SemaphoreType.DMA,
            pltpu.SemaphoreType.DMA,
        ],
    )
    def k(table_hbm, idx_hbm, out_hbm, idx_vm, rb0, rb1, si0, sg0, sg1, sw0, sw1):
        wid = lax.axis_index("s") * NC + lax.axis_index("c")
        base_batch = wid * batches_per_worker

        bufs = (rb0, rb1)
        gsems = (sg0, sg1)
        wsems = (sw0, sw1)

        def idx_load(j, t):
            pltpu.make_async_copy(
                idx_hbm.at[wid, j], idx_vm.at[t], si0
            ).start()

        def idx_wait(t):
            pltpu.make_async_copy(
                idx_hbm.at[wid, 0], idx_vm.at[t], si0
            ).wait()

        def gather_chunk(t):
            @pl.loop(0, CHUNK, step=16)
            def _(r):
                v = idx_vm[t, pl.ds(r, 16)]
                for l in range(16):
                    pltpu.make_async_copy(
                        table_hbm.at[pl.ds(v[l], 1)],
                        bufs[t].at[pl.ds(r + l, 1)],
                        gsems[t],
                    ).start()

        def gather_drain(t):
            pltpu.make_async_copy(
                table_hbm.at[pl.ds(0, CHUNK)], bufs[t], gsems[t]
            ).wait()

        def write_start(j, t):
            pltpu.make_async_copy(
                bufs[t].reshape(BPC, SEQ, HIDDEN),
                out_hbm.at[pl.ds(base_batch + j * BPC, BPC)],
                wsems[t],
            ).start()

        def write_wait(t):
            pltpu.make_async_copy(
                bufs[t].reshape(BPC, SEQ, HIDDEN),
                out_hbm.at[pl.ds(base_batch, BPC)],
                wsems[t],
            ).wait()

        idx_load(0, 0)
        idx_load(1, 1)

        @pl.loop(0, nb, step=2)
        def _(j):
            for t in range(2):
                idx_wait(t)

                @pl.when(j > 0)
                def _():
                    write_wait(t)

                gather_chunk(t)

            for t in range(2):
                gather_drain(t)
                write_start(j + t, t)

            @pl.when(j + 2 < nb)
            def _():
                for t in range(2):
                    idx_load(j + 2 + t, t)

        for t in range(2):
            write_wait(t)

    return k(table, idx3)


def kernel(input_ids, table):
    n_batch, seq = input_ids.shape
    idx3 = input_ids.reshape(NW, n_batch * seq // (NW * CHUNK), CHUNK).astype(
        jnp.int32
    )
    return _gather_call(table, idx3, n_batch)
